# Initial kernel scaffold; baseline (speedup 1.0000x reference)
import jax
import jax.numpy as jnp
from jax.experimental import pallas as pl


def kernel(x, positions, Wq, bq, Wk, bk, Wv, bv, Wo, bo, g1, b1, g2, b2, Wm1, bm1, Wm2, bm2):
    raise NotImplementedError("write your pallas kernel here")


# speedup vs baseline: 3.9154x; 3.9154x over previous
import jax
import jax.numpy as jnp
from jax.experimental import pallas as pl
from jax.experimental.pallas import tpu as pltpu

N = 4096
DIM = 512
H = 8
DH = DIM // H
K = 32
MLP_HIDDEN = int(DIM * 4.0)
SCALE = DH ** -0.5

BLK = 256
NBLK = N // BLK


def _ln(x, g, b, eps=1e-5):
    mu = jnp.mean(x, axis=-1, keepdims=True)
    var = jnp.mean((x - mu) ** 2, axis=-1, keepdims=True)
    return (x - mu) / jnp.sqrt(var + eps) * g + b


def _qkv_body(x_ref, g1_ref, b1_ref, wqt_ref, bq_ref, wkt_ref, bk_ref,
              wvt_ref, bv_ref, q_ref, k_ref, v_ref):
    h = _ln(x_ref[...], g1_ref[...], b1_ref[...])
    q_ref[...] = jnp.dot(h, wqt_ref[...],
                         preferred_element_type=jnp.float32, precision=jax.lax.Precision.HIGHEST) + bq_ref[...]
    k_ref[...] = jnp.dot(h, wkt_ref[...],
                         preferred_element_type=jnp.float32, precision=jax.lax.Precision.HIGHEST) + bk_ref[...]
    v_ref[...] = jnp.dot(h, wvt_ref[...],
                         preferred_element_type=jnp.float32, precision=jax.lax.Precision.HIGHEST) + bv_ref[...]


def _mask_body(pb_ref, pf_ref, mask_ref):
    i = pl.program_id(0)
    pb = pb_ref[...]
    pf = pf_ref[...]
    sqb = jnp.sum(pb * pb, axis=1, keepdims=True)
    sqf = jnp.sum(pf * pf, axis=1, keepdims=True).T
    dots = jax.lax.dot_general(pb.astype(jnp.bfloat16), pf.astype(jnp.bfloat16),
                               (((1,), (1,)), ((), ())),
                               preferred_element_type=jnp.float32)
    d = sqb + sqf - 2.0 * dots
    ib = jax.lax.bitcast_convert_type(d, jnp.int32)
    ukey = jnp.where(ib >= 0, ib, jnp.int32(-2147483648) + (-1 - ib))

    def step(_, lohi):
        lo, hi = lohi
        mid = lo + ((hi - lo) >> 1)
        cnt = jnp.sum((ukey <= mid).astype(jnp.int32), axis=1, keepdims=True)
        ge = cnt >= K + 1
        return jnp.where(ge, lo, mid + 1), jnp.where(ge, mid, hi)

    lo0 = jnp.full((BLK, 1), jnp.int32(-1036831950))
    hi0 = jnp.full((BLK, 1), jnp.int32(0x40800000))
    lo, hi = jax.lax.fori_loop(0, 32, step, (lo0, hi0))
    m0 = jnp.min(ukey, axis=1, keepdims=True)
    mask_ref[...] = ((ukey <= hi) & (ukey > m0)).astype(jnp.int8)


def _attn_body(q_ref, kt_ref, v_ref, mask_ref, wot_ref, bo_ref, x_ref,
               out_ref):
    q = q_ref[...]
    neg = jnp.float32(-1e30)
    msk = mask_ref[...] != 0
    heads = []
    for h in range(H):
        s = jnp.dot(q[:, h * DH:(h + 1) * DH], kt_ref[h * DH:(h + 1) * DH, :],
                    preferred_element_type=jnp.float32, precision=jax.lax.Precision.HIGHEST)
        s = jnp.where(msk, s * SCALE, neg)
        m = jnp.max(s, axis=1, keepdims=True)
        p = jnp.exp(s - m)
        w = p / jnp.sum(p, axis=1, keepdims=True)
        heads.append(jnp.dot(w, v_ref[:, h * DH:(h + 1) * DH],
                             preferred_element_type=jnp.float32, precision=jax.lax.Precision.HIGHEST))
    o = jnp.concatenate(heads, axis=1)
    out_ref[...] = jnp.dot(o, wot_ref[...],
                           preferred_element_type=jnp.float32, precision=jax.lax.Precision.HIGHEST) \
        + bo_ref[...] + x_ref[...]


def _mlp_body(x1_ref, g2_ref, b2_ref, wm1t_ref, bm1_ref, wm2t_ref, bm2_ref,
              out_ref):
    x1 = x1_ref[...]
    h2 = _ln(x1, g2_ref[...], b2_ref[...])
    a = jnp.dot(h2, wm1t_ref[...],
                preferred_element_type=jnp.float32, precision=jax.lax.Precision.HIGHEST) + bm1_ref[...]
    a = 0.5 * a * (1.0 + jax.lax.erf(a * jnp.float32(0.7071067811865476)))
    m = jnp.dot(a, wm2t_ref[...],
                preferred_element_type=jnp.float32, precision=jax.lax.Precision.HIGHEST) + bm2_ref[...]
    out_ref[...] = x1 + m


def _row_spec(width):
    return pl.BlockSpec((BLK, width), lambda i: (i, 0))


def _full_spec(shape):
    nd = len(shape)
    return pl.BlockSpec(shape, lambda i: (0,) * nd)


def kernel(x, positions, Wq, bq, Wk, bk, Wv, bv, Wo, bo, g1, b1, g2, b2,
           Wm1, bm1, Wm2, bm2):
    p8 = jnp.pad(positions, ((0, 0), (0, 5)))
    g1r = g1.reshape(1, DIM)
    b1r = b1.reshape(1, DIM)
    g2r = g2.reshape(1, DIM)
    b2r = b2.reshape(1, DIM)

    q, k, v = pl.pallas_call(
        _qkv_body,
        grid=(NBLK,),
        in_specs=[_row_spec(DIM), _full_spec((1, DIM)), _full_spec((1, DIM)),
                  _full_spec((DIM, DIM)), _full_spec((1, DIM)),
                  _full_spec((DIM, DIM)), _full_spec((1, DIM)),
                  _full_spec((DIM, DIM)), _full_spec((1, DIM))],
        out_specs=[_row_spec(DIM)] * 3,
        out_shape=[jax.ShapeDtypeStruct((N, DIM), jnp.float32)] * 3,
    )(x, g1r, b1r, Wq.T, bq.reshape(1, DIM), Wk.T, bk.reshape(1, DIM),
      Wv.T, bv.reshape(1, DIM))

    mask = pl.pallas_call(
        _mask_body,
        grid=(NBLK,),
        in_specs=[_row_spec(8), _full_spec((N, 8))],
        out_specs=_row_spec(N),
        out_shape=jax.ShapeDtypeStruct((N, N), jnp.int8),
    )(p8, p8)

    x1 = pl.pallas_call(
        _attn_body,
        grid=(NBLK,),
        in_specs=[_row_spec(DIM), _full_spec((DIM, N)), _full_spec((N, DIM)),
                  _row_spec(N), _full_spec((DIM, DIM)), _full_spec((1, DIM)),
                  _row_spec(DIM)],
        out_specs=_row_spec(DIM),
        out_shape=jax.ShapeDtypeStruct((N, DIM), jnp.float32),
    )(q, k.T, v, mask, Wo.T, bo.reshape(1, DIM), x)

    out = pl.pallas_call(
        _mlp_body,
        grid=(NBLK,),
        in_specs=[_row_spec(DIM), _full_spec((1, DIM)), _full_spec((1, DIM)),
                  _full_spec((DIM, MLP_HIDDEN)), _full_spec((1, MLP_HIDDEN)),
                  _full_spec((MLP_HIDDEN, DIM)), _full_spec((1, DIM))],
        out_specs=_row_spec(DIM),
        out_shape=jax.ShapeDtypeStruct((N, DIM), jnp.float32),
    )(x1, g2r, b2r, Wm1.T, bm1.reshape(1, MLP_HIDDEN), Wm2.T,
      bm2.reshape(1, DIM))
    return out


# speedup vs baseline: 5.6521x; 1.4435x over previous
import jax
import jax.numpy as jnp
from jax.experimental import pallas as pl
from jax.experimental.pallas import tpu as pltpu

N = 4096
DIM = 512
H = 8
DH = DIM // H
K = 32
MLP_HIDDEN = int(DIM * 4.0)
SCALE = DH ** -0.5

BLK = 256
NBLK = N // BLK


def _dot3(a, b):
    ahi = a.astype(jnp.bfloat16)
    alo = (a - ahi.astype(jnp.float32)).astype(jnp.bfloat16)
    bhi = b.astype(jnp.bfloat16)
    blo = (b - bhi.astype(jnp.float32)).astype(jnp.bfloat16)
    f = jnp.float32
    return (jnp.dot(ahi, bhi, preferred_element_type=f)
            + (jnp.dot(ahi, blo, preferred_element_type=f)
               + jnp.dot(alo, bhi, preferred_element_type=f)))

def _ln(x, g, b, eps=1e-5):
    mu = jnp.mean(x, axis=-1, keepdims=True)
    var = jnp.mean((x - mu) ** 2, axis=-1, keepdims=True)
    return (x - mu) / jnp.sqrt(var + eps) * g + b


def _qkv_body(x_ref, g1_ref, b1_ref, wqt_ref, bq_ref, wkt_ref, bk_ref,
              wvt_ref, bv_ref, q_ref, k_ref, v_ref):
    h = _ln(x_ref[...], g1_ref[...], b1_ref[...])
    q_ref[...] = _dot3(h, wqt_ref[...]) + bq_ref[...]
    k_ref[...] = _dot3(h, wkt_ref[...]) + bk_ref[...]
    v_ref[...] = _dot3(h, wvt_ref[...]) + bv_ref[...]


def _mask_body(pb_ref, pf_ref, mask_ref):
    i = pl.program_id(0)
    pb = pb_ref[...]
    pf = pf_ref[...]
    sqb = jnp.sum(pb * pb, axis=1, keepdims=True)
    sqf = jnp.sum(pf * pf, axis=1, keepdims=True).T
    dots = jax.lax.dot_general(pb.astype(jnp.bfloat16), pf.astype(jnp.bfloat16),
                               (((1,), (1,)), ((), ())),
                               preferred_element_type=jnp.float32)
    d = sqb + sqf - 2.0 * dots
    ib = jax.lax.bitcast_convert_type(d, jnp.int32)
    ukey = jnp.where(ib >= 0, ib, jnp.int32(-2147483648) + (-1 - ib))

    def step(_, lohi):
        lo, hi = lohi
        mid = lo + ((hi - lo) >> 1)
        cnt = jnp.sum((ukey <= mid).astype(jnp.int32), axis=1, keepdims=True)
        ge = cnt >= K + 1
        return jnp.where(ge, lo, mid + 1), jnp.where(ge, mid, hi)

    lo0 = jnp.full((BLK, 1), jnp.int32(-1036831950))
    hi0 = jnp.full((BLK, 1), jnp.int32(0x40800000))
    lo, hi = jax.lax.fori_loop(0, 32, step, (lo0, hi0))
    m0 = jnp.min(ukey, axis=1, keepdims=True)
    mask_ref[...] = ((ukey <= hi) & (ukey > m0)).astype(jnp.int8)


def _attn_body(q_ref, kt_ref, v_ref, mask_ref, wot_ref, bo_ref, x_ref,
               out_ref):
    q = q_ref[...]
    neg = jnp.float32(-1e30)
    msk = mask_ref[...] != 0
    heads = []
    for h in range(H):
        s = _dot3(q[:, h * DH:(h + 1) * DH], kt_ref[h * DH:(h + 1) * DH, :])
        s = jnp.where(msk, s * SCALE, neg)
        m = jnp.max(s, axis=1, keepdims=True)
        p = jnp.exp(s - m)
        w = p / jnp.sum(p, axis=1, keepdims=True)
        heads.append(_dot3(w, v_ref[:, h * DH:(h + 1) * DH]))
    o = jnp.concatenate(heads, axis=1)
    out_ref[...] = _dot3(o, wot_ref[...]) \
        + bo_ref[...] + x_ref[...]


def _mlp_body(x1_ref, g2_ref, b2_ref, wm1t_ref, bm1_ref, wm2t_ref, bm2_ref,
              out_ref):
    x1 = x1_ref[...]
    h2 = _ln(x1, g2_ref[...], b2_ref[...])
    a = _dot3(h2, wm1t_ref[...]) + bm1_ref[...]
    a = 0.5 * a * (1.0 + jax.lax.erf(a * jnp.float32(0.7071067811865476)))
    m = _dot3(a, wm2t_ref[...]) + bm2_ref[...]
    out_ref[...] = x1 + m


def _row_spec(width):
    return pl.BlockSpec((BLK, width), lambda i: (i, 0))


def _full_spec(shape):
    nd = len(shape)
    return pl.BlockSpec(shape, lambda i: (0,) * nd)


def kernel(x, positions, Wq, bq, Wk, bk, Wv, bv, Wo, bo, g1, b1, g2, b2,
           Wm1, bm1, Wm2, bm2):
    p8 = jnp.pad(positions, ((0, 0), (0, 5)))
    g1r = g1.reshape(1, DIM)
    b1r = b1.reshape(1, DIM)
    g2r = g2.reshape(1, DIM)
    b2r = b2.reshape(1, DIM)

    q, k, v = pl.pallas_call(
        _qkv_body,
        grid=(NBLK,),
        in_specs=[_row_spec(DIM), _full_spec((1, DIM)), _full_spec((1, DIM)),
                  _full_spec((DIM, DIM)), _full_spec((1, DIM)),
                  _full_spec((DIM, DIM)), _full_spec((1, DIM)),
                  _full_spec((DIM, DIM)), _full_spec((1, DIM))],
        out_specs=[_row_spec(DIM)] * 3,
        out_shape=[jax.ShapeDtypeStruct((N, DIM), jnp.float32)] * 3,
    )(x, g1r, b1r, Wq.T, bq.reshape(1, DIM), Wk.T, bk.reshape(1, DIM),
      Wv.T, bv.reshape(1, DIM))

    mask = pl.pallas_call(
        _mask_body,
        grid=(NBLK,),
        in_specs=[_row_spec(8), _full_spec((N, 8))],
        out_specs=_row_spec(N),
        out_shape=jax.ShapeDtypeStruct((N, N), jnp.int8),
    )(p8, p8)

    x1 = pl.pallas_call(
        _attn_body,
        grid=(NBLK,),
        in_specs=[_row_spec(DIM), _full_spec((DIM, N)), _full_spec((N, DIM)),
                  _row_spec(N), _full_spec((DIM, DIM)), _full_spec((1, DIM)),
                  _row_spec(DIM)],
        out_specs=_row_spec(DIM),
        out_shape=jax.ShapeDtypeStruct((N, DIM), jnp.float32),
    )(q, k.T, v, mask, Wo.T, bo.reshape(1, DIM), x)

    out = pl.pallas_call(
        _mlp_body,
        grid=(NBLK,),
        in_specs=[_row_spec(DIM), _full_spec((1, DIM)), _full_spec((1, DIM)),
                  _full_spec((DIM, MLP_HIDDEN)), _full_spec((1, MLP_HIDDEN)),
                  _full_spec((MLP_HIDDEN, DIM)), _full_spec((1, DIM))],
        out_specs=_row_spec(DIM),
        out_shape=jax.ShapeDtypeStruct((N, DIM), jnp.float32),
    )(x1, g2r, b2r, Wm1.T, bm1.reshape(1, MLP_HIDDEN), Wm2.T,
      bm2.reshape(1, DIM))
    return out


# speedup vs baseline: 9.6620x; 1.7095x over previous
import jax
import jax.numpy as jnp
from jax.experimental import pallas as pl
from jax.experimental.pallas import tpu as pltpu

N = 4096
DIM = 512
H = 8
DH = DIM // H
K = 32
MLP_HIDDEN = int(DIM * 4.0)
SCALE = DH ** -0.5
LOG2E = 1.4426950408889634

BLK = 256
NBLK = N // BLK

_bf = jnp.bfloat16
_f32 = jnp.float32


def _dot(a, b):
    return jnp.dot(a.astype(_bf), b.astype(_bf), preferred_element_type=_f32)


def _dot3(a, b):
    ahi = a.astype(_bf)
    alo = (a - ahi.astype(_f32)).astype(_bf)
    bhi = b.astype(_bf)
    blo = (b - bhi.astype(_f32)).astype(_bf)
    return (jnp.dot(ahi, bhi, preferred_element_type=_f32)
            + (jnp.dot(ahi, blo, preferred_element_type=_f32)
               + jnp.dot(alo, bhi, preferred_element_type=_f32)))


def _ln(x, g, b, eps=1e-5):
    mu = jnp.mean(x, axis=-1, keepdims=True)
    var = jnp.mean((x - mu) ** 2, axis=-1, keepdims=True)
    return (x - mu) / jnp.sqrt(var + eps) * g + b


def _qkv_body(x_ref, g1_ref, b1_ref, wqt_ref, bq_ref, wkt_ref, bk_ref,
              wvt_ref, bv_ref, q_ref, k_ref, v_ref):
    h = _ln(x_ref[...], g1_ref[...], b1_ref[...])
    q_ref[...] = _dot3(h, wqt_ref[...]) + bq_ref[...]
    k_ref[...] = _dot3(h, wkt_ref[...]) + bk_ref[...]
    v_ref[...] = _dot3(h, wvt_ref[...]) + bv_ref[...]


def _attn_body(pb_ref, pf_ref, ones_ref, q_ref, kt_ref, v_ref, wot_ref,
               bo_ref, x_ref, out_ref):
    pb = pb_ref[...]
    pf = pf_ref[...]
    sqb = jnp.sum(pb * pb, axis=1, keepdims=True)
    sqf = jnp.sum(pf * pf, axis=1, keepdims=True).T
    dots = jax.lax.dot_general(pb.astype(_bf), pf.astype(_bf),
                               (((1,), (1,)), ((), ())),
                               preferred_element_type=_f32)
    d = sqb + sqf - 2.0 * dots
    ib = jax.lax.bitcast_convert_type(d, jnp.int32)
    ukey = jnp.where(ib >= 0, ib, jnp.int32(-2147483648) + (-1 - ib))
    ones_col = ones_ref[...]

    def step(_, lohi):
        lo, hi = lohi
        mid = lo + ((hi - lo) >> 1)
        ind = jnp.where(ukey <= mid, _f32(1.0), _f32(0.0)).astype(_bf)
        cnt = jnp.dot(ind, ones_col,
                      preferred_element_type=_f32)[:, :1]
        ge = cnt >= _f32(K + 1)
        return jnp.where(ge, lo, mid + 1), jnp.where(ge, mid, hi)

    lo0 = jnp.full((BLK, 1), jnp.int32(-1036831950))
    hi0 = jnp.full((BLK, 1), jnp.int32(0x40800000))
    lo, hi = jax.lax.fori_loop(0, 32, step, (lo0, hi0))
    m0 = jnp.min(ukey, axis=1, keepdims=True)
    msk = (ukey <= hi) & (ukey > m0)

    q = q_ref[...]
    heads = []
    for h in range(H):
        s = _dot(q[:, h * DH:(h + 1) * DH], kt_ref[h * DH:(h + 1) * DH, :])
        p = jnp.where(msk, jnp.exp2(s * _f32(SCALE * LOG2E)), _f32(0.0))
        denom = jnp.sum(p, axis=1, keepdims=True)
        oh = _dot(p, v_ref[:, h * DH:(h + 1) * DH]) / denom
        heads.append(oh)
    o = jnp.concatenate(heads, axis=1)
    out_ref[...] = _dot(o, wot_ref[...]) + bo_ref[...] + x_ref[...]


def _mlp_body(x1_ref, g2_ref, b2_ref, wm1t_ref, bm1_ref, wm2t_ref, bm2_ref,
              out_ref):
    x1 = x1_ref[...]
    h2 = _ln(x1, g2_ref[...], b2_ref[...])
    a = _dot(h2, wm1t_ref[...]) + bm1_ref[...]
    a = 0.5 * a * (1.0 + jax.lax.erf(a * _f32(0.7071067811865476)))
    m = _dot(a, wm2t_ref[...]) + bm2_ref[...]
    out_ref[...] = x1 + m


def _row_spec(width):
    return pl.BlockSpec((BLK, width), lambda i: (i, 0))


def _full_spec(shape):
    nd = len(shape)
    return pl.BlockSpec(shape, lambda i: (0,) * nd)


def kernel(x, positions, Wq, bq, Wk, bk, Wv, bv, Wo, bo, g1, b1, g2, b2,
           Wm1, bm1, Wm2, bm2):
    p8 = jnp.pad(positions, ((0, 0), (0, 5)))
    ones_col = jnp.ones((N, 128), jnp.bfloat16)

    q, k, v = pl.pallas_call(
        _qkv_body,
        grid=(NBLK,),
        in_specs=[_row_spec(DIM), _full_spec((1, DIM)), _full_spec((1, DIM)),
                  _full_spec((DIM, DIM)), _full_spec((1, DIM)),
                  _full_spec((DIM, DIM)), _full_spec((1, DIM)),
                  _full_spec((DIM, DIM)), _full_spec((1, DIM))],
        out_specs=[_row_spec(DIM)] * 3,
        out_shape=[jax.ShapeDtypeStruct((N, DIM), jnp.float32)] * 3,
    )(x, g1.reshape(1, DIM), b1.reshape(1, DIM),
      Wq.T, bq.reshape(1, DIM), Wk.T, bk.reshape(1, DIM),
      Wv.T, bv.reshape(1, DIM))

    x1 = pl.pallas_call(
        _attn_body,
        grid=(NBLK,),
        in_specs=[_row_spec(8), _full_spec((N, 8)), _full_spec((N, 128)),
                  _row_spec(DIM), _full_spec((DIM, N)), _full_spec((N, DIM)),
                  _full_spec((DIM, DIM)), _full_spec((1, DIM)),
                  _row_spec(DIM)],
        out_specs=_row_spec(DIM),
        out_shape=jax.ShapeDtypeStruct((N, DIM), jnp.float32),
    )(p8, p8, ones_col, q, k.T, v, Wo.T, bo.reshape(1, DIM), x)

    out = pl.pallas_call(
        _mlp_body,
        grid=(NBLK,),
        in_specs=[_row_spec(DIM), _full_spec((1, DIM)), _full_spec((1, DIM)),
                  _full_spec((DIM, MLP_HIDDEN)), _full_spec((1, MLP_HIDDEN)),
                  _full_spec((MLP_HIDDEN, DIM)), _full_spec((1, DIM))],
        out_specs=_row_spec(DIM),
        out_shape=jax.ShapeDtypeStruct((N, DIM), jnp.float32),
    )(x1, g2.reshape(1, DIM), b2.reshape(1, DIM),
      Wm1.T, bm1.reshape(1, MLP_HIDDEN), Wm2.T, bm2.reshape(1, DIM))
    return out


# speedup vs baseline: 9.7792x; 1.0121x over previous
import jax
import jax.numpy as jnp
from jax.experimental import pallas as pl
from jax.experimental.pallas import tpu as pltpu

N = 4096
DIM = 512
H = 8
DH = DIM // H
K = 32
MLP_HIDDEN = int(DIM * 4.0)
SCALE = DH ** -0.5
LOG2E = 1.4426950408889634

BLK = 256
NBLK = N // BLK

_bf = jnp.bfloat16
_f32 = jnp.float32


def _dot(a, b):
    return jnp.dot(a.astype(_bf), b.astype(_bf), preferred_element_type=_f32)


def _dot3(a, b):
    ahi = a.astype(_bf)
    alo = (a - ahi.astype(_f32)).astype(_bf)
    bhi = b.astype(_bf)
    blo = (b - bhi.astype(_f32)).astype(_bf)
    return (jnp.dot(ahi, bhi, preferred_element_type=_f32)
            + (jnp.dot(ahi, blo, preferred_element_type=_f32)
               + jnp.dot(alo, bhi, preferred_element_type=_f32)))


def _ln(x, g, b, eps=1e-5):
    mu = jnp.mean(x, axis=-1, keepdims=True)
    var = jnp.mean((x - mu) ** 2, axis=-1, keepdims=True)
    return (x - mu) / jnp.sqrt(var + eps) * g + b


def _qkv_body(x_ref, g1_ref, b1_ref, wqt_ref, bq_ref, wkt_ref, bk_ref,
              wvt_ref, bv_ref, q_ref, k_ref, v_ref):
    h = _ln(x_ref[...], g1_ref[...], b1_ref[...])
    q_ref[...] = _dot(h, wqt_ref[...]) + bq_ref[...]
    k_ref[...] = _dot(h, wkt_ref[...]) + bk_ref[...]
    v_ref[...] = _dot(h, wvt_ref[...]) + bv_ref[...]


def _attn_body(pb_ref, pf_ref, ones_ref, q_ref, kt_ref, v_ref, wot_ref,
               bo_ref, x_ref, out_ref):
    pb = pb_ref[...]
    pf = pf_ref[...]
    sqb = jnp.sum(pb * pb, axis=1, keepdims=True)
    sqf = jnp.sum(pf * pf, axis=1, keepdims=True).T
    dots = jax.lax.dot_general(pb.astype(_bf), pf.astype(_bf),
                               (((1,), (1,)), ((), ())),
                               preferred_element_type=_f32)
    d = sqb + sqf - 2.0 * dots
    ib = jax.lax.bitcast_convert_type(d, jnp.int32)
    ukey = jnp.where(ib >= 0, ib, jnp.int32(-2147483648) + (-1 - ib))
    ones_col = ones_ref[...]

    def step(_, lohi):
        lo, hi = lohi
        mid = lo + ((hi - lo) >> 1)
        ind = jnp.where(ukey <= mid, _f32(1.0), _f32(0.0)).astype(_bf)
        cnt = jnp.dot(ind, ones_col,
                      preferred_element_type=_f32)[:, :1]
        ge = cnt >= _f32(K + 1)
        return jnp.where(ge, lo, mid + 1), jnp.where(ge, mid, hi)

    lo0 = jnp.full((BLK, 1), jnp.int32(-1036831950))
    hi0 = jnp.full((BLK, 1), jnp.int32(0x40800000))
    lo, hi = jax.lax.fori_loop(0, 32, step, (lo0, hi0))
    m0 = jnp.min(ukey, axis=1, keepdims=True)
    msk = (ukey <= hi) & (ukey > m0)

    q = q_ref[...]
    heads = []
    for h in range(H):
        s = _dot(q[:, h * DH:(h + 1) * DH], kt_ref[h * DH:(h + 1) * DH, :])
        p = jnp.where(msk, jnp.exp2(s * _f32(SCALE * LOG2E)), _f32(0.0))
        denom = jnp.sum(p, axis=1, keepdims=True)
        oh = _dot(p, v_ref[:, h * DH:(h + 1) * DH]) / denom
        heads.append(oh)
    o = jnp.concatenate(heads, axis=1)
    out_ref[...] = _dot(o, wot_ref[...]) + bo_ref[...] + x_ref[...]


def _mlp_body(x1_ref, g2_ref, b2_ref, wm1t_ref, bm1_ref, wm2t_ref, bm2_ref,
              out_ref):
    x1 = x1_ref[...]
    h2 = _ln(x1, g2_ref[...], b2_ref[...])
    a = _dot(h2, wm1t_ref[...]) + bm1_ref[...]
    a = 0.5 * a * (1.0 + jax.lax.erf(a * _f32(0.7071067811865476)))
    m = _dot(a, wm2t_ref[...]) + bm2_ref[...]
    out_ref[...] = x1 + m


def _row_spec(width):
    return pl.BlockSpec((BLK, width), lambda i: (i, 0))


def _full_spec(shape):
    nd = len(shape)
    return pl.BlockSpec(shape, lambda i: (0,) * nd)


def kernel(x, positions, Wq, bq, Wk, bk, Wv, bv, Wo, bo, g1, b1, g2, b2,
           Wm1, bm1, Wm2, bm2):
    p8 = jnp.pad(positions, ((0, 0), (0, 5)))
    ones_col = jnp.ones((N, 128), jnp.bfloat16)

    q, k, v = pl.pallas_call(
        _qkv_body,
        grid=(NBLK,),
        in_specs=[_row_spec(DIM), _full_spec((1, DIM)), _full_spec((1, DIM)),
                  _full_spec((DIM, DIM)), _full_spec((1, DIM)),
                  _full_spec((DIM, DIM)), _full_spec((1, DIM)),
                  _full_spec((DIM, DIM)), _full_spec((1, DIM))],
        out_specs=[_row_spec(DIM)] * 3,
        out_shape=[jax.ShapeDtypeStruct((N, DIM), jnp.float32)] * 3,
    )(x, g1.reshape(1, DIM), b1.reshape(1, DIM),
      Wq.T, bq.reshape(1, DIM), Wk.T, bk.reshape(1, DIM),
      Wv.T, bv.reshape(1, DIM))

    x1 = pl.pallas_call(
        _attn_body,
        grid=(NBLK,),
        in_specs=[_row_spec(8), _full_spec((N, 8)), _full_spec((N, 128)),
                  _row_spec(DIM), _full_spec((DIM, N)), _full_spec((N, DIM)),
                  _full_spec((DIM, DIM)), _full_spec((1, DIM)),
                  _row_spec(DIM)],
        out_specs=_row_spec(DIM),
        out_shape=jax.ShapeDtypeStruct((N, DIM), jnp.float32),
    )(p8, p8, ones_col, q, k.T, v, Wo.T, bo.reshape(1, DIM), x)

    out = pl.pallas_call(
        _mlp_body,
        grid=(NBLK,),
        in_specs=[_row_spec(DIM), _full_spec((1, DIM)), _full_spec((1, DIM)),
                  _full_spec((DIM, MLP_HIDDEN)), _full_spec((1, MLP_HIDDEN)),
                  _full_spec((MLP_HIDDEN, DIM)), _full_spec((1, DIM))],
        out_specs=_row_spec(DIM),
        out_shape=jax.ShapeDtypeStruct((N, DIM), jnp.float32),
    )(x1, g2.reshape(1, DIM), b2.reshape(1, DIM),
      Wm1.T, bm1.reshape(1, MLP_HIDDEN), Wm2.T, bm2.reshape(1, DIM))
    return out


# speedup vs baseline: 11.5279x; 1.1788x over previous
import jax
import jax.numpy as jnp
from jax.experimental import pallas as pl
from jax.experimental.pallas import tpu as pltpu

N = 4096
DIM = 512
H = 8
DH = DIM // H
K = 32
MLP_HIDDEN = int(DIM * 4.0)
SCALE = DH ** -0.5
LOG2E = 1.4426950408889634

BLK = 256
NBLK = N // BLK

_bf = jnp.bfloat16
_f32 = jnp.float32


def _dot(a, b):
    return jnp.dot(a.astype(_bf), b.astype(_bf), preferred_element_type=_f32)


def _dot3(a, b):
    ahi = a.astype(_bf)
    alo = (a - ahi.astype(_f32)).astype(_bf)
    bhi = b.astype(_bf)
    blo = (b - bhi.astype(_f32)).astype(_bf)
    return (jnp.dot(ahi, bhi, preferred_element_type=_f32)
            + (jnp.dot(ahi, blo, preferred_element_type=_f32)
               + jnp.dot(alo, bhi, preferred_element_type=_f32)))


def _ln(x, g, b, eps=1e-5):
    mu = jnp.mean(x, axis=-1, keepdims=True)
    var = jnp.mean((x - mu) ** 2, axis=-1, keepdims=True)
    return (x - mu) / jnp.sqrt(var + eps) * g + b


def _qkv_body(x_ref, g1_ref, b1_ref, wqt_ref, bq_ref, wkt_ref, bk_ref,
              wvt_ref, bv_ref, q_ref, k_ref, v_ref):
    h = _ln(x_ref[...], g1_ref[...], b1_ref[...])
    q_ref[...] = _dot(h, wqt_ref[...]) + bq_ref[...]
    k_ref[...] = _dot(h, wkt_ref[...]) + bk_ref[...]
    v_ref[...] = _dot(h, wvt_ref[...]) + bv_ref[...]


def _attn_body(pb_ref, pf_ref, ones_ref, q_ref, kt_ref, v_ref, wot_ref,
               bo_ref, x_ref, out_ref):
    pb = pb_ref[...]
    pf = pf_ref[...]
    sqb = jnp.sum(pb * pb, axis=1, keepdims=True)
    sqf = jnp.sum(pf * pf, axis=1, keepdims=True).T
    dots = jax.lax.dot_general(pb.astype(_bf), pf.astype(_bf),
                               (((1,), (1,)), ((), ())),
                               preferred_element_type=_f32)
    d = sqb + sqf - 2.0 * dots
    ib = jax.lax.bitcast_convert_type(d, jnp.int32)
    ukey = jnp.where(ib >= 0, ib, jnp.int32(-2147483648) + (-1 - ib))
    ones_col = ones_ref[...]

    def step(_, lohi):
        lo, hi = lohi
        mid = lo + ((hi - lo) >> 1)
        cnt = jnp.sum((ukey <= mid).astype(jnp.int32), axis=1, keepdims=True)
        ge = cnt >= K + 1
        return jnp.where(ge, lo, mid + 1), jnp.where(ge, mid, hi)

    lo0 = jnp.full((BLK, 1), jnp.int32(-1036831950))
    hi0 = jnp.full((BLK, 1), jnp.int32(0x40800000))
    lo, hi = jax.lax.fori_loop(0, 32, step, (lo0, hi0))
    m0 = jnp.min(ukey, axis=1, keepdims=True)
    msk = (ukey <= hi) & (ukey > m0)

    q = q_ref[...]
    heads = []
    for h in range(H):
        s = _dot(q[:, h * DH:(h + 1) * DH], kt_ref[h * DH:(h + 1) * DH, :])
        p = jnp.where(msk, jnp.exp2(s * _f32(SCALE * LOG2E)), _f32(0.0))
        denom = jnp.sum(p, axis=1, keepdims=True)
        oh = _dot(p, v_ref[:, h * DH:(h + 1) * DH]) / denom
        heads.append(oh)
    o = jnp.concatenate(heads, axis=1)
    out_ref[...] = _dot(o, wot_ref[...]) + bo_ref[...] + x_ref[...]


def _mlp_body(x1_ref, g2_ref, b2_ref, wm1t_ref, bm1_ref, wm2t_ref, bm2_ref,
              out_ref):
    x1 = x1_ref[...]
    h2 = _ln(x1, g2_ref[...], b2_ref[...])
    a = _dot(h2, wm1t_ref[...]) + bm1_ref[...]
    a = 0.5 * a * (1.0 + jax.lax.erf(a * _f32(0.7071067811865476)))
    m = _dot(a, wm2t_ref[...]) + bm2_ref[...]
    out_ref[...] = x1 + m


def _row_spec(width):
    return pl.BlockSpec((BLK, width), lambda i: (i, 0))


def _full_spec(shape):
    nd = len(shape)
    return pl.BlockSpec(shape, lambda i: (0,) * nd)


def kernel(x, positions, Wq, bq, Wk, bk, Wv, bv, Wo, bo, g1, b1, g2, b2,
           Wm1, bm1, Wm2, bm2):
    p8 = jnp.pad(positions, ((0, 0), (0, 5)))
    ones_col = jnp.ones((N, 128), jnp.bfloat16)

    q, k, v = pl.pallas_call(
        _qkv_body,
        grid=(NBLK,),
        in_specs=[_row_spec(DIM), _full_spec((1, DIM)), _full_spec((1, DIM)),
                  _full_spec((DIM, DIM)), _full_spec((1, DIM)),
                  _full_spec((DIM, DIM)), _full_spec((1, DIM)),
                  _full_spec((DIM, DIM)), _full_spec((1, DIM))],
        out_specs=[_row_spec(DIM)] * 3,
        out_shape=[jax.ShapeDtypeStruct((N, DIM), jnp.float32)] * 3,
    )(x, g1.reshape(1, DIM), b1.reshape(1, DIM),
      Wq.T, bq.reshape(1, DIM), Wk.T, bk.reshape(1, DIM),
      Wv.T, bv.reshape(1, DIM))

    x1 = pl.pallas_call(
        _attn_body,
        grid=(NBLK,),
        in_specs=[_row_spec(8), _full_spec((N, 8)), _full_spec((N, 128)),
                  _row_spec(DIM), _full_spec((DIM, N)), _full_spec((N, DIM)),
                  _full_spec((DIM, DIM)), _full_spec((1, DIM)),
                  _row_spec(DIM)],
        out_specs=_row_spec(DIM),
        out_shape=jax.ShapeDtypeStruct((N, DIM), jnp.float32),
    )(p8, p8, ones_col, q, k.T, v, Wo.T, bo.reshape(1, DIM), x)

    out = pl.pallas_call(
        _mlp_body,
        grid=(NBLK,),
        in_specs=[_row_spec(DIM), _full_spec((1, DIM)), _full_spec((1, DIM)),
                  _full_spec((DIM, MLP_HIDDEN)), _full_spec((1, MLP_HIDDEN)),
                  _full_spec((MLP_HIDDEN, DIM)), _full_spec((1, DIM))],
        out_specs=_row_spec(DIM),
        out_shape=jax.ShapeDtypeStruct((N, DIM), jnp.float32),
    )(x1, g2.reshape(1, DIM), b2.reshape(1, DIM),
      Wm1.T, bm1.reshape(1, MLP_HIDDEN), Wm2.T, bm2.reshape(1, DIM))
    return out


# speedup vs baseline: 11.8674x; 1.0295x over previous
import jax
import jax.numpy as jnp
from jax.experimental import pallas as pl
from jax.experimental.pallas import tpu as pltpu

N = 4096
DIM = 512
H = 8
DH = DIM // H
K = 32
MLP_HIDDEN = int(DIM * 4.0)
SCALE = DH ** -0.5
LOG2E = 1.4426950408889634

BLK = 256
NBLK = N // BLK

_bf = jnp.bfloat16
_f32 = jnp.float32


def _dot(a, b):
    return jnp.dot(a.astype(_bf), b.astype(_bf), preferred_element_type=_f32)


def _dot3(a, b):
    ahi = a.astype(_bf)
    alo = (a - ahi.astype(_f32)).astype(_bf)
    bhi = b.astype(_bf)
    blo = (b - bhi.astype(_f32)).astype(_bf)
    return (jnp.dot(ahi, bhi, preferred_element_type=_f32)
            + (jnp.dot(ahi, blo, preferred_element_type=_f32)
               + jnp.dot(alo, bhi, preferred_element_type=_f32)))


def _ln(x, g, b, eps=1e-5):
    mu = jnp.mean(x, axis=-1, keepdims=True)
    var = jnp.mean((x - mu) ** 2, axis=-1, keepdims=True)
    return (x - mu) / jnp.sqrt(var + eps) * g + b


def _qkv_body(x_ref, g1_ref, b1_ref, wqt_ref, bq_ref, wkt_ref, bk_ref,
              wvt_ref, bv_ref, q_ref, k_ref, v_ref):
    h = _ln(x_ref[...], g1_ref[...], b1_ref[...])
    q_ref[...] = (_dot(h, wqt_ref[...]) + bq_ref[...]).astype(_bf)
    k_ref[...] = (_dot(h, wkt_ref[...]) + bk_ref[...]).astype(_bf)
    v_ref[...] = (_dot(h, wvt_ref[...]) + bv_ref[...]).astype(_bf)


def _attn_body(pb_ref, pf_ref, ones_ref, q_ref, kt_ref, v_ref, wot_ref,
               bo_ref, x_ref, out_ref):
    pb = pb_ref[...]
    pf = pf_ref[...]
    sqb = jnp.sum(pb * pb, axis=1, keepdims=True)
    sqf = jnp.sum(pf * pf, axis=1, keepdims=True).T
    dots = jax.lax.dot_general(pb.astype(_bf), pf.astype(_bf),
                               (((1,), (1,)), ((), ())),
                               preferred_element_type=_f32)
    d = sqb + sqf - 2.0 * dots
    ib = jax.lax.bitcast_convert_type(d, jnp.int32)
    ukey = jnp.where(ib >= 0, ib, jnp.int32(-2147483648) + (-1 - ib))
    ones_col = ones_ref[...]

    def step(_, lohi):
        lo, hi = lohi
        mid = lo + ((hi - lo) >> 1)
        cnt = jnp.sum((ukey <= mid).astype(jnp.int32), axis=1, keepdims=True)
        ge = cnt >= K + 1
        return jnp.where(ge, lo, mid + 1), jnp.where(ge, mid, hi)

    lo0 = jnp.full((BLK, 1), jnp.int32(-1036831950))
    hi0 = jnp.full((BLK, 1), jnp.int32(0x40800000))
    lo, hi = jax.lax.fori_loop(0, 32, step, (lo0, hi0))
    m0 = jnp.min(ukey, axis=1, keepdims=True)
    msk = (ukey <= hi) & (ukey > m0)

    q = q_ref[...]
    heads = []
    for h in range(H):
        s = _dot(q[:, h * DH:(h + 1) * DH], kt_ref[h * DH:(h + 1) * DH, :])
        p = jnp.where(msk, jnp.exp2(s * _f32(SCALE * LOG2E)), _f32(0.0))
        denom = jnp.sum(p, axis=1, keepdims=True)
        oh = _dot(p, v_ref[:, h * DH:(h + 1) * DH]) / denom
        heads.append(oh)
    o = jnp.concatenate(heads, axis=1)
    out_ref[...] = _dot(o, wot_ref[...]) + bo_ref[...] + x_ref[...]


def _mlp_body(x1_ref, g2_ref, b2_ref, wm1t_ref, bm1_ref, wm2t_ref, bm2_ref,
              out_ref):
    x1 = x1_ref[...]
    h2 = _ln(x1, g2_ref[...], b2_ref[...])
    a = _dot(h2, wm1t_ref[...]) + bm1_ref[...]
    a = 0.5 * a * (1.0 + jax.lax.erf(a * _f32(0.7071067811865476)))
    m = _dot(a, wm2t_ref[...]) + bm2_ref[...]
    out_ref[...] = x1 + m


def _row_spec(width):
    return pl.BlockSpec((BLK, width), lambda i: (i, 0))


def _full_spec(shape):
    nd = len(shape)
    return pl.BlockSpec(shape, lambda i: (0,) * nd)


def kernel(x, positions, Wq, bq, Wk, bk, Wv, bv, Wo, bo, g1, b1, g2, b2,
           Wm1, bm1, Wm2, bm2):
    p8 = jnp.pad(positions, ((0, 0), (0, 5)))
    ones_col = jnp.ones((N, 128), jnp.bfloat16)

    q, k, v = pl.pallas_call(
        _qkv_body,
        grid=(NBLK,),
        in_specs=[_row_spec(DIM), _full_spec((1, DIM)), _full_spec((1, DIM)),
                  _full_spec((DIM, DIM)), _full_spec((1, DIM)),
                  _full_spec((DIM, DIM)), _full_spec((1, DIM)),
                  _full_spec((DIM, DIM)), _full_spec((1, DIM))],
        out_specs=[_row_spec(DIM)] * 3,
        out_shape=[jax.ShapeDtypeStruct((N, DIM), jnp.bfloat16)] * 3,
    )(x, g1.reshape(1, DIM), b1.reshape(1, DIM),
      Wq.T.astype(jnp.bfloat16), bq.reshape(1, DIM),
      Wk.T.astype(jnp.bfloat16), bk.reshape(1, DIM),
      Wv.T.astype(jnp.bfloat16), bv.reshape(1, DIM))

    x1 = pl.pallas_call(
        _attn_body,
        grid=(NBLK,),
        in_specs=[_row_spec(8), _full_spec((N, 8)), _full_spec((N, 128)),
                  _row_spec(DIM), _full_spec((DIM, N)), _full_spec((N, DIM)),
                  _full_spec((DIM, DIM)), _full_spec((1, DIM)),
                  _row_spec(DIM)],
        out_specs=_row_spec(DIM),
        out_shape=jax.ShapeDtypeStruct((N, DIM), jnp.float32),
    )(p8, p8, ones_col, q, k.T, v, Wo.T.astype(jnp.bfloat16), bo.reshape(1, DIM), x)

    out = pl.pallas_call(
        _mlp_body,
        grid=(NBLK,),
        in_specs=[_row_spec(DIM), _full_spec((1, DIM)), _full_spec((1, DIM)),
                  _full_spec((DIM, MLP_HIDDEN)), _full_spec((1, MLP_HIDDEN)),
                  _full_spec((MLP_HIDDEN, DIM)), _full_spec((1, DIM))],
        out_specs=_row_spec(DIM),
        out_shape=jax.ShapeDtypeStruct((N, DIM), jnp.float32),
    )(x1, g2.reshape(1, DIM), b2.reshape(1, DIM),
      Wm1.T.astype(jnp.bfloat16), bm1.reshape(1, MLP_HIDDEN),
      Wm2.T.astype(jnp.bfloat16), bm2.reshape(1, DIM))
    return out


# speedup vs baseline: 12.3797x; 1.0432x over previous
import jax
import jax.numpy as jnp
from jax.experimental import pallas as pl
from jax.experimental.pallas import tpu as pltpu

N = 4096
DIM = 512
H = 8
DH = DIM // H
K = 32
MLP_HIDDEN = int(DIM * 4.0)
SCALE = DH ** -0.5
LOG2E = 1.4426950408889634

BLK = 256
NBLK = N // BLK

_bf = jnp.bfloat16
_f32 = jnp.float32


def _dot(a, b):
    return jnp.dot(a.astype(_bf), b.astype(_bf), preferred_element_type=_f32)


def _dot3(a, b):
    ahi = a.astype(_bf)
    alo = (a - ahi.astype(_f32)).astype(_bf)
    bhi = b.astype(_bf)
    blo = (b - bhi.astype(_f32)).astype(_bf)
    return (jnp.dot(ahi, bhi, preferred_element_type=_f32)
            + (jnp.dot(ahi, blo, preferred_element_type=_f32)
               + jnp.dot(alo, bhi, preferred_element_type=_f32)))


def _ln(x, g, b, eps=1e-5):
    mu = jnp.mean(x, axis=-1, keepdims=True)
    var = jnp.mean((x - mu) ** 2, axis=-1, keepdims=True)
    return (x - mu) / jnp.sqrt(var + eps) * g + b


def _qkv_body(x_ref, g1_ref, b1_ref, wqt_ref, bq_ref, wkt_ref, bk_ref,
              wvt_ref, bv_ref, q_ref, k_ref, v_ref):
    h = _ln(x_ref[...], g1_ref[...], b1_ref[...])
    q_ref[...] = ((_dot(h, wqt_ref[...]) + bq_ref[...])
                  * _f32(SCALE * LOG2E)).astype(_bf)
    k_ref[...] = (_dot(h, wkt_ref[...]) + bk_ref[...]).astype(_bf)
    v_ref[...] = (_dot(h, wvt_ref[...]) + bv_ref[...]).astype(_bf)


def _attn_body(pb_ref, pf_ref, ones_ref, q_ref, kt_ref, v_ref, wot_ref,
               bo_ref, x_ref, out_ref):
    pb = pb_ref[...]
    pf = pf_ref[...]
    sqb = jnp.sum(pb * pb, axis=1, keepdims=True)
    sqf = jnp.sum(pf * pf, axis=1, keepdims=True).T
    dots = jax.lax.dot_general(pb.astype(_bf), pf.astype(_bf),
                               (((1,), (1,)), ((), ())),
                               preferred_element_type=_f32)
    d = sqb + sqf - 2.0 * dots
    ib = jax.lax.bitcast_convert_type(d, jnp.int32)
    ukey = jnp.where(ib >= 0, ib, jnp.int32(-2147483648) + (-1 - ib))
    ones_col = ones_ref[...]

    def step(_, lohi):
        lo, hi = lohi
        mid = lo + ((hi - lo) >> 1)
        cnt = jnp.sum((ukey <= mid).astype(jnp.int32), axis=1, keepdims=True)
        ge = cnt >= K + 1
        return jnp.where(ge, lo, mid + 1), jnp.where(ge, mid, hi)

    lo0 = jnp.full((BLK, 1), jnp.int32(-1036831950))
    hi0 = jnp.full((BLK, 1), jnp.int32(0x40800000))
    lo, hi = jax.lax.fori_loop(0, 31, step, (lo0, hi0))
    m0 = jnp.min(ukey, axis=1, keepdims=True)
    msk = (ukey <= hi) & (ukey > m0)

    q = q_ref[...]
    heads = []
    for h in range(H):
        s = _dot(q[:, h * DH:(h + 1) * DH], kt_ref[h * DH:(h + 1) * DH, :])
        p = jnp.where(msk, jnp.exp2(s), _f32(0.0))
        denom = jnp.sum(p, axis=1, keepdims=True)
        oh = _dot(p, v_ref[:, h * DH:(h + 1) * DH]) / denom
        heads.append(oh)
    o = jnp.concatenate(heads, axis=1)
    out_ref[...] = _dot(o, wot_ref[...]) + bo_ref[...] + x_ref[...]


def _mlp_body(x1_ref, g2_ref, b2_ref, wm1t_ref, bm1_ref, wm2t_ref, bm2_ref,
              out_ref):
    x1 = x1_ref[...]
    h2 = _ln(x1, g2_ref[...], b2_ref[...])
    a = _dot(h2, wm1t_ref[...]) + bm1_ref[...]
    a = 0.5 * a * (1.0 + jax.lax.erf(a * _f32(0.7071067811865476)))
    m = _dot(a, wm2t_ref[...]) + bm2_ref[...]
    out_ref[...] = x1 + m


def _row_spec(width):
    return pl.BlockSpec((BLK, width), lambda i: (i, 0))


def _full_spec(shape):
    nd = len(shape)
    return pl.BlockSpec(shape, lambda i: (0,) * nd)


def kernel(x, positions, Wq, bq, Wk, bk, Wv, bv, Wo, bo, g1, b1, g2, b2,
           Wm1, bm1, Wm2, bm2):
    p8 = jnp.pad(positions, ((0, 0), (0, 5)))
    ones_col = jnp.ones((N, 128), jnp.bfloat16)

    q, k, v = pl.pallas_call(
        _qkv_body,
        grid=(NBLK,),
        in_specs=[_row_spec(DIM), _full_spec((1, DIM)), _full_spec((1, DIM)),
                  _full_spec((DIM, DIM)), _full_spec((1, DIM)),
                  _full_spec((DIM, DIM)), _full_spec((1, DIM)),
                  _full_spec((DIM, DIM)), _full_spec((1, DIM))],
        out_specs=[_row_spec(DIM)] * 3,
        out_shape=[jax.ShapeDtypeStruct((N, DIM), jnp.bfloat16)] * 3,
    )(x, g1.reshape(1, DIM), b1.reshape(1, DIM),
      Wq.T.astype(jnp.bfloat16), bq.reshape(1, DIM),
      Wk.T.astype(jnp.bfloat16), bk.reshape(1, DIM),
      Wv.T.astype(jnp.bfloat16), bv.reshape(1, DIM))

    x1 = pl.pallas_call(
        _attn_body,
        grid=(NBLK,),
        in_specs=[_row_spec(8), _full_spec((N, 8)), _full_spec((N, 128)),
                  _row_spec(DIM), _full_spec((DIM, N)), _full_spec((N, DIM)),
                  _full_spec((DIM, DIM)), _full_spec((1, DIM)),
                  _row_spec(DIM)],
        out_specs=_row_spec(DIM),
        out_shape=jax.ShapeDtypeStruct((N, DIM), jnp.float32),
    )(p8, p8, ones_col, q, k.T, v, Wo.T.astype(jnp.bfloat16), bo.reshape(1, DIM), x)

    out = pl.pallas_call(
        _mlp_body,
        grid=(NBLK,),
        in_specs=[_row_spec(DIM), _full_spec((1, DIM)), _full_spec((1, DIM)),
                  _full_spec((DIM, MLP_HIDDEN)), _full_spec((1, MLP_HIDDEN)),
                  _full_spec((MLP_HIDDEN, DIM)), _full_spec((1, DIM))],
        out_specs=_row_spec(DIM),
        out_shape=jax.ShapeDtypeStruct((N, DIM), jnp.float32),
    )(x1, g2.reshape(1, DIM), b2.reshape(1, DIM),
      Wm1.T.astype(jnp.bfloat16), bm1.reshape(1, MLP_HIDDEN),
      Wm2.T.astype(jnp.bfloat16), bm2.reshape(1, DIM))
    return out
